# baseline (device time: 71497 ns/iter reference)
import jax
import jax.numpy as jnp
from jax import lax
from jax.experimental import pallas as pl
from jax.experimental.pallas import tpu as pltpu

Y = 4
N_HOPS = Y - 1
S = 2
N_BANDS = 4
DLANES = 128


def kernel(x, W):
    t, d = x.shape
    _, c = W.shape
    v = Y * c
    tb = t // N_BANDS
    sc = c // S
    hc = sc // 2

    def body(x_ref, w_ref, out_ref, comm_ref, dsum_ref,
             y_s, y_r, xd_s, xd_r, zd_s, zd_r, xf_s, xf_r, zf_s, zf_r,
             dn_s, dn_r):
        my_x = lax.axis_index("x")
        my_y = lax.axis_index("y")
        my_z = lax.axis_index("z")
        left = (my_y - 1) % Y
        right = (my_y + 1) % Y
        zb = my_z % 2
        pz_z = my_z + 1 - 2 * zb
        ox = 1 - my_x
        r_mine = 2 * my_x + zb
        r_px = 2 * ox + zb
        r_pz = 2 * my_x + (1 - zb)
        r_d = 2 * ox + (1 - zb)

        DEV_RIGHT = (my_x, right, my_z)
        DEV_PX = (ox, my_y, my_z)
        DEV_PZ = (my_x, my_y, pz_z)

        barrier = pltpu.get_barrier_semaphore()
        for dev in ((my_x, left, my_z), DEV_RIGHT, DEV_PX, DEV_PZ):
            pl.semaphore_signal(
                barrier, inc=1, device_id=dev,
                device_id_type=pl.DeviceIdType.MESH,
            )
        pl.semaphore_wait(barrier, 4)

        def piece(band, origin, s):
            return comm_ref.at[
                pl.ds(band * tb, tb), pl.ds(origin * c + s * sc, sc)
            ]

        def half(band, origin, s, h):
            return comm_ref.at[
                pl.ds(band * tb, tb), pl.ds(origin * c + s * sc + h * hc, hc)
            ]

        def rdma(src, ss, rr, dev):
            return pltpu.make_async_remote_copy(
                src_ref=src, dst_ref=src, send_sem=ss, recv_sem=rr,
                device_id=dev, device_id_type=pl.DeviceIdType.MESH,
            )

        d_full = jnp.zeros((t, 1), jnp.float32)
        sends = []
        x_bf = x_ref[:, :].astype(jnp.bfloat16)
        for s in range(S):
            e_sub = jnp.exp(
                jnp.dot(
                    x_bf,
                    w_ref[:, pl.ds(s * sc, sc)].astype(jnp.bfloat16),
                    preferred_element_type=jnp.float32,
                )
            )
            comm_ref[:, pl.ds(my_y * c + s * sc, sc)] = e_sub.astype(jnp.bfloat16)
            rd0 = rdma(piece(r_mine, my_y, s), y_s.at[0, s], y_r.at[0, s], DEV_RIGHT)
            rd0.start()
            sends.append(rd0)
            d_full = d_full + jnp.sum(e_sub, axis=1, keepdims=True)

        dsum_ref[0, :, :] = jnp.broadcast_to(d_full, (t, DLANES))
        denom = d_full
        for h in range(N_HOPS):
            dr = pltpu.make_async_remote_copy(
                src_ref=dsum_ref.at[h],
                dst_ref=dsum_ref.at[h + 1],
                send_sem=dn_s.at[h], recv_sem=dn_r.at[h],
                device_id=DEV_RIGHT, device_id_type=pl.DeviceIdType.MESH,
            )
            dr.start()
            dr.wait()
            denom = denom + dsum_ref[h + 1, :, :1]
        inv = 1.0 / denom
        dsum_ref[0, :, :] = jnp.broadcast_to(inv, (t, DLANES))
        inv_mine = dsum_ref[pl.ds(0, 1), pl.ds(r_mine * tb, tb), :1][0]
        inv_px = dsum_ref[pl.ds(0, 1), pl.ds(r_px * tb, tb), :1][0]
        inv_pz = dsum_ref[pl.ds(0, 1), pl.ds(r_pz * tb, tb), :1][0]
        inv_d = dsum_ref[pl.ds(0, 1), pl.ds(r_d * tb, tb), :1][0]

        for s in range(S):
            sl = pl.ds(my_y * c + s * sc, sc)
            out_ref[:, sl] = comm_ref[:, sl].astype(jnp.float32) * inv

        def store(band, origin, s, ncols, inv_band):
            csl = pl.ds(origin * c + s * sc, ncols)
            rsl = pl.ds(band * tb, tb)
            out_ref[rsl, csl] = (
                comm_ref[rsl, csl].astype(jnp.float32) * inv_band
            )

        for ev in range(N_HOPS):
            origin = (my_y - ev - 1) % Y
            for s in range(S):
                rdma(piece(r_mine, origin, s), y_s.at[ev, s], y_r.at[ev, s],
                     DEV_RIGHT).wait_recv()
                if ev + 1 < N_HOPS:
                    f = rdma(piece(r_mine, origin, s),
                             y_s.at[ev + 1, s], y_r.at[ev + 1, s], DEV_RIGHT)
                    f.start()
                    sends.append(f)
                a = rdma(piece(r_mine, origin, s),
                         xd_s.at[ev, s], xd_r.at[ev, s], DEV_PX)
                a.start()
                sends.append(a)
                b = rdma(piece(r_mine, origin, s),
                         zd_s.at[ev, s], zd_r.at[ev, s], DEV_PZ)
                b.start()
                sends.append(b)
                store(r_mine, origin, s, sc, inv_mine)

        for ev in range(N_HOPS):
            origin = (my_y - ev - 1) % Y
            for s in range(S):
                rdma(piece(r_px, origin, s), xd_s.at[ev, s], xd_r.at[ev, s],
                     DEV_PX).wait_recv()
                g = rdma(half(r_px, origin, s, zb),
                         zf_s.at[ev, s], zf_r.at[ev, s], DEV_PZ)
                g.start()
                sends.append(g)
                store(r_px, origin, s, sc, inv_px)

                rdma(piece(r_pz, origin, s), zd_s.at[ev, s], zd_r.at[ev, s],
                     DEV_PZ).wait_recv()
                h = rdma(half(r_pz, origin, s, zb),
                         xf_s.at[ev, s], xf_r.at[ev, s], DEV_PX)
                h.start()
                sends.append(h)
                store(r_pz, origin, s, sc, inv_pz)

        for ev in range(N_HOPS):
            origin = (my_y - ev - 1) % Y
            for s in range(S):
                rdma(half(r_d, origin, s, zb), xf_s.at[ev, s], xf_r.at[ev, s],
                     DEV_PX).wait_recv()
                rdma(half(r_d, origin, s, 1 - zb), zf_s.at[ev, s],
                     zf_r.at[ev, s], DEV_PZ).wait_recv()
                store(r_d, origin, s, sc, inv_d)

        for rr in sends:
            rr.wait_send()

    return pl.pallas_call(
        body,
        out_shape=jax.ShapeDtypeStruct((t, v), jnp.float32),
        in_specs=[
            pl.BlockSpec(memory_space=pltpu.VMEM),
            pl.BlockSpec(memory_space=pltpu.VMEM),
        ],
        out_specs=pl.BlockSpec(memory_space=pltpu.VMEM),
        scratch_shapes=[
            pltpu.VMEM((t, v), jnp.bfloat16),
            pltpu.VMEM((Y, t, DLANES), jnp.float32),
            pltpu.SemaphoreType.DMA((N_HOPS, S)),
            pltpu.SemaphoreType.DMA((N_HOPS, S)),
            pltpu.SemaphoreType.DMA((N_HOPS, S)),
            pltpu.SemaphoreType.DMA((N_HOPS, S)),
            pltpu.SemaphoreType.DMA((N_HOPS, S)),
            pltpu.SemaphoreType.DMA((N_HOPS, S)),
            pltpu.SemaphoreType.DMA((N_HOPS, S)),
            pltpu.SemaphoreType.DMA((N_HOPS, S)),
            pltpu.SemaphoreType.DMA((N_HOPS, S)),
            pltpu.SemaphoreType.DMA((N_HOPS, S)),
            pltpu.SemaphoreType.DMA((N_HOPS,)),
            pltpu.SemaphoreType.DMA((N_HOPS,)),
        ],
        compiler_params=pltpu.CompilerParams(collective_id=3),
    )(x, W)


# device time: 63171 ns/iter; 1.1318x vs baseline; 1.1318x over previous
import jax
import jax.numpy as jnp
from jax import lax
from jax.experimental import pallas as pl
from jax.experimental.pallas import tpu as pltpu

Y = 4
N_HOPS = Y - 1
S = 2
N_BANDS = 4


def kernel(x, W):
    t, d = x.shape
    _, c = W.shape
    v = Y * c
    tb = t // N_BANDS
    sc = c // S
    hc = sc // 2

    def body(x_ref, w_ref, out_ref, comm_ref,
             y_s, y_r, xd_s, xd_r, zd_s, zd_r, xf_s, xf_r, zf_s, zf_r):
        my_x = lax.axis_index("x")
        my_y = lax.axis_index("y")
        my_z = lax.axis_index("z")
        left = (my_y - 1) % Y
        right = (my_y + 1) % Y
        zb = my_z % 2
        pz_z = my_z + 1 - 2 * zb
        ox = 1 - my_x
        r_mine = 2 * my_x + zb
        r_px = 2 * ox + zb
        r_pz = 2 * my_x + (1 - zb)
        r_d = 2 * ox + (1 - zb)

        DEV_RIGHT = (my_x, right, my_z)
        DEV_PX = (ox, my_y, my_z)
        DEV_PZ = (my_x, my_y, pz_z)

        barrier = pltpu.get_barrier_semaphore()
        for dev in ((my_x, left, my_z), DEV_RIGHT, DEV_PX, DEV_PZ):
            pl.semaphore_signal(
                barrier, inc=1, device_id=dev,
                device_id_type=pl.DeviceIdType.MESH,
            )
        pl.semaphore_wait(barrier, 4)

        def piece(band, origin, s):
            return comm_ref.at[
                pl.ds(band * tb, tb), pl.ds(origin * c + s * sc, sc)
            ]

        def half(band, origin, s, h):
            return comm_ref.at[
                pl.ds(band * tb, tb), pl.ds(origin * c + s * sc + h * hc, hc)
            ]

        def rdma(src, ss, rr, dev):
            return pltpu.make_async_remote_copy(
                src_ref=src, dst_ref=src, send_sem=ss, recv_sem=rr,
                device_id=dev, device_id_type=pl.DeviceIdType.MESH,
            )

        d_full = jnp.zeros((t, 1), jnp.float32)
        sends = []
        x_bf = x_ref[:, :].astype(jnp.bfloat16)
        for s in range(S):
            e_sub = jnp.exp(
                jnp.dot(
                    x_bf,
                    w_ref[:, pl.ds(s * sc, sc)].astype(jnp.bfloat16),
                    preferred_element_type=jnp.float32,
                )
            )
            comm_ref[:, pl.ds(my_y * c + s * sc, sc)] = e_sub.astype(jnp.bfloat16)
            rd0 = rdma(piece(r_mine, my_y, s), y_s.at[0, s], y_r.at[0, s], DEV_RIGHT)
            rd0.start()
            sends.append(rd0)
            d_full = d_full + jnp.sum(e_sub, axis=1, keepdims=True)

        def bandsum(band, origin, s, ncols):
            return jnp.sum(
                comm_ref[
                    pl.ds(band * tb, tb), pl.ds(origin * c + s * sc, ncols)
                ].astype(jnp.float32),
                axis=1, keepdims=True,
            )

        a_mine = jnp.zeros((tb, 1), jnp.float32)
        a_px = jnp.zeros((tb, 1), jnp.float32)
        a_pz = jnp.zeros((tb, 1), jnp.float32)
        a_d = jnp.zeros((tb, 1), jnp.float32)
        for ev in range(N_HOPS):
            origin = (my_y - ev - 1) % Y
            for s in range(S):
                rdma(piece(r_mine, origin, s), y_s.at[ev, s], y_r.at[ev, s],
                     DEV_RIGHT).wait_recv()
                if ev + 1 < N_HOPS:
                    f = rdma(piece(r_mine, origin, s),
                             y_s.at[ev + 1, s], y_r.at[ev + 1, s], DEV_RIGHT)
                    f.start()
                    sends.append(f)
                a = rdma(piece(r_mine, origin, s),
                         xd_s.at[ev, s], xd_r.at[ev, s], DEV_PX)
                a.start()
                sends.append(a)
                b = rdma(piece(r_mine, origin, s),
                         zd_s.at[ev, s], zd_r.at[ev, s], DEV_PZ)
                b.start()
                sends.append(b)
                a_mine = a_mine + bandsum(r_mine, origin, s, sc)

        for ev in range(N_HOPS):
            origin = (my_y - ev - 1) % Y
            for s in range(S):
                rdma(piece(r_px, origin, s), xd_s.at[ev, s], xd_r.at[ev, s],
                     DEV_PX).wait_recv()
                g = rdma(half(r_px, origin, s, zb),
                         zf_s.at[ev, s], zf_r.at[ev, s], DEV_PZ)
                g.start()
                sends.append(g)
                a_px = a_px + bandsum(r_px, origin, s, sc)

                rdma(piece(r_pz, origin, s), zd_s.at[ev, s], zd_r.at[ev, s],
                     DEV_PZ).wait_recv()
                h = rdma(half(r_pz, origin, s, zb),
                         xf_s.at[ev, s], xf_r.at[ev, s], DEV_PX)
                h.start()
                sends.append(h)
                a_pz = a_pz + bandsum(r_pz, origin, s, sc)

        for ev in range(N_HOPS):
            origin = (my_y - ev - 1) % Y
            for s in range(S):
                rdma(half(r_d, origin, s, zb), xf_s.at[ev, s], xf_r.at[ev, s],
                     DEV_PX).wait_recv()
                rdma(half(r_d, origin, s, 1 - zb), zf_s.at[ev, s],
                     zf_r.at[ev, s], DEV_PZ).wait_recv()
                a_d = a_d + bandsum(r_d, origin, s, sc)

        band_accs = []
        for b in range(N_BANDS):
            xb, zbb = b // 2, b % 2
            band_accs.append(
                jnp.where(
                    xb == my_x,
                    jnp.where(zbb == zb, a_mine, a_pz),
                    jnp.where(zbb == zb, a_px, a_d),
                )
            )
        denom = d_full + jnp.concatenate(band_accs, axis=0)
        inv = 1.0 / denom
        out_ref[:, :] = comm_ref[:, :].astype(jnp.float32) * inv

        for rr in sends:
            rr.wait_send()

    return pl.pallas_call(
        body,
        out_shape=jax.ShapeDtypeStruct((t, v), jnp.float32),
        in_specs=[
            pl.BlockSpec(memory_space=pltpu.VMEM),
            pl.BlockSpec(memory_space=pltpu.VMEM),
        ],
        out_specs=pl.BlockSpec(memory_space=pltpu.VMEM),
        scratch_shapes=[
            pltpu.VMEM((t, v), jnp.bfloat16),
            pltpu.SemaphoreType.DMA((N_HOPS, S)),
            pltpu.SemaphoreType.DMA((N_HOPS, S)),
            pltpu.SemaphoreType.DMA((N_HOPS, S)),
            pltpu.SemaphoreType.DMA((N_HOPS, S)),
            pltpu.SemaphoreType.DMA((N_HOPS, S)),
            pltpu.SemaphoreType.DMA((N_HOPS, S)),
            pltpu.SemaphoreType.DMA((N_HOPS, S)),
            pltpu.SemaphoreType.DMA((N_HOPS, S)),
            pltpu.SemaphoreType.DMA((N_HOPS, S)),
            pltpu.SemaphoreType.DMA((N_HOPS, S)),
        ],
        compiler_params=pltpu.CompilerParams(collective_id=3),
    )(x, W)
